# baseline (speedup 1.0000x reference)
import functools

import jax
import jax.numpy as jnp
from jax import lax
from jax.experimental import pallas as pl
from jax.experimental.pallas import tpu as pltpu
from jax.experimental.pallas import tpu_sc as plsc

_NUM_EMB = 8192
_EMB_DIM = 64
_K_CHUNK = 1024


def _conv2d(x, w, b, stride):
    y = lax.conv_general_dilated(
        x.astype(jnp.bfloat16), w.astype(jnp.bfloat16), (stride, stride),
        ((1, 1), (1, 1)), dimension_numbers=('NCHW', 'OIHW', 'NCHW'),
        preferred_element_type=jnp.float32)
    return y + b[None, :, None, None]


def _conv2d_t(x, w, b):
    wt = jnp.flip(w, axis=(2, 3)).transpose(1, 0, 2, 3)
    y = lax.conv_general_dilated(
        x, wt, (1, 1), ((2, 2), (2, 2)), lhs_dilation=(2, 2),
        dimension_numbers=('NCHW', 'OIHW', 'NCHW'))
    return y + b[None, :, None, None]


def _vq_body(z_ref, cb_ref, idx_ref, loss_ref):
    i = pl.program_id(0)
    bm = z_ref.shape[0]
    z = z_ref[...]
    zs = jnp.sum(z * z, axis=1, keepdims=True)
    z_bf = z.astype(jnp.bfloat16)
    ids = lax.broadcasted_iota(jnp.int32, (bm, _K_CHUNK), 1)

    best_val = jnp.full((bm, 1), jnp.inf, jnp.float32)
    best_idx = jnp.zeros((bm, 1), jnp.int32)
    for j in range(_NUM_EMB // _K_CHUNK):
        c = cb_ref[pl.ds(j * _K_CHUNK, _K_CHUNK), :]
        cs = jnp.sum(c * c, axis=1, keepdims=True)
        scores = cs.T - 2.0 * lax.dot_general(
            z_bf, c.astype(jnp.bfloat16), (((1,), (1,)), ((), ())),
            preferred_element_type=jnp.float32)
        local_min = jnp.min(scores, axis=1, keepdims=True)
        cand = jnp.where(scores == local_min, ids, jnp.int32(2**30))
        local_arg = jnp.min(cand, axis=1, keepdims=True) + j * _K_CHUNK
        upd = local_min < best_val
        best_val = jnp.where(upd, local_min, best_val)
        best_idx = jnp.where(upd, local_arg, best_idx)

    idx_ref[0, 0, :] = best_idx[:, 0]
    total = jnp.sum(best_val + zs).reshape(1, 1)
    prev = jnp.where(i == 0, jnp.zeros((1, 1), jnp.float32), loss_ref[...])
    loss_ref[...] = prev + total


def _vq_argmin(z_flat, codebook):
    m = z_flat.shape[0]
    n_blocks = 8
    bm = m // n_blocks
    idx3, loss = pl.pallas_call(
        _vq_body,
        grid=(n_blocks,),
        in_specs=[
            pl.BlockSpec((bm, _EMB_DIM), lambda i: (i, 0)),
            pl.BlockSpec((_NUM_EMB, _EMB_DIM), lambda i: (0, 0)),
        ],
        out_specs=[
            pl.BlockSpec((1, 1, bm), lambda i: (i, 0, 0)),
            pl.BlockSpec((1, 1), lambda i: (0, 0)),
        ],
        out_shape=[
            jax.ShapeDtypeStruct((n_blocks, 1, bm), jnp.int32),
            jax.ShapeDtypeStruct((1, 1), jnp.float32),
        ],
    )(z_flat, codebook)
    return idx3.reshape(m), loss[0, 0]


def _sc_gather(codebook, idx_padded, n_padded):
    n_workers = 32
    rows_per_w = n_padded // n_workers
    mesh = plsc.VectorSubcoreMesh(core_axis_name="c", subcore_axis_name="s")

    @functools.partial(
        pl.kernel,
        out_type=jax.ShapeDtypeStruct((n_padded, _EMB_DIM), jnp.float32),
        mesh=mesh,
        scratch_types=[
            pltpu.VMEM((rows_per_w,), jnp.int32),
            pltpu.VMEM((rows_per_w, _EMB_DIM), jnp.float32),
            pltpu.SemaphoreType.DMA,
        ],
        compiler_params=pltpu.CompilerParams(use_tc_tiling_on_sc=False),
    )
    def gather_kernel(table_hbm, idx_hbm, out_hbm, idx_v, rows_v, sem):
        wid = lax.axis_index("s") * 2 + lax.axis_index("c")
        base = wid * rows_per_w
        pltpu.sync_copy(idx_hbm.at[pl.ds(base, rows_per_w)], idx_v)
        pltpu.async_copy(table_hbm.at[idx_v], rows_v, sem).wait()
        pltpu.sync_copy(rows_v, out_hbm.at[pl.ds(base, rows_per_w)])

    return gather_kernel(codebook, idx_padded)


def kernel(x, enc_w1, enc_b1, enc_w2, enc_b2, enc_w3, enc_b3, codebook,
           dec_w1, dec_b1, dec_w2, dec_b2, dec_w3, dec_b3):
    z = jax.nn.relu(_conv2d(x, enc_w1, enc_b1, 2))
    z = jax.nn.relu(_conv2d(z, enc_w2, enc_b2, 2))
    z_e = _conv2d(z, enc_w3, enc_b3, 2)
    B, C, H, W = z_e.shape
    m = B * H * W
    z_flat = z_e.reshape(B, C, H * W).transpose(0, 2, 1).reshape(m, C)

    indices, loss_sum = _vq_argmin(z_flat, codebook)

    n_padded = ((m + 255) // 256) * 256
    idx_padded = jnp.concatenate(
        [indices, jnp.zeros((n_padded - m,), jnp.int32)])
    z_q_flat = _sc_gather(codebook, idx_padded, n_padded)[:m]

    z_q = z_q_flat.reshape(B, H * W, C).transpose(0, 2, 1).reshape(B, C, H, W)

    r = jax.nn.relu(_conv2d_t(z_q, dec_w1, dec_b1))
    r = jax.nn.relu(_conv2d_t(r, dec_w2, dec_b2))
    x_recon = jax.nn.sigmoid(_conv2d_t(r, dec_w3, dec_b3))

    loss = 1.25 * loss_sum / jnp.float32(m * C)
    return (x_recon, loss)


# speedup vs baseline: 1.0082x; 1.0082x over previous
import functools

import jax
import jax.numpy as jnp
from jax import lax
from jax.experimental import pallas as pl
from jax.experimental.pallas import tpu as pltpu
from jax.experimental.pallas import tpu_sc as plsc

_NUM_EMB = 8192
_EMB_DIM = 64
_K_CHUNK = 1024


def _conv2d(x, w, b, stride):
    y = lax.conv_general_dilated(
        x.astype(jnp.bfloat16),
        w.transpose(2, 3, 1, 0).astype(jnp.bfloat16), (stride, stride),
        ((1, 1), (1, 1)), dimension_numbers=('NHWC', 'HWIO', 'NHWC'),
        preferred_element_type=jnp.float32)
    return y + b[None, None, None, :]


def _conv2d_t(x, w, b):
    wt = jnp.flip(w, axis=(2, 3)).transpose(2, 3, 0, 1)
    y = lax.conv_general_dilated(
        x, wt, (1, 1), ((2, 2), (2, 2)), lhs_dilation=(2, 2),
        dimension_numbers=('NHWC', 'HWIO', 'NHWC'))
    return y + b[None, None, None, :]


def _vq_body(z_ref, cb_ref, idx_ref, loss_ref):
    i = pl.program_id(0)
    bm = z_ref.shape[0]
    z = z_ref[...]
    zs = jnp.sum(z * z, axis=1, keepdims=True)
    z_bf = z.astype(jnp.bfloat16)
    ids = lax.broadcasted_iota(jnp.int32, (bm, _K_CHUNK), 1)

    best_val = jnp.full((bm, 1), jnp.inf, jnp.float32)
    best_idx = jnp.zeros((bm, 1), jnp.int32)
    for j in range(_NUM_EMB // _K_CHUNK):
        c = cb_ref[pl.ds(j * _K_CHUNK, _K_CHUNK), :]
        cs = jnp.sum(c * c, axis=1, keepdims=True)
        scores = cs.T - 2.0 * lax.dot_general(
            z_bf, c.astype(jnp.bfloat16), (((1,), (1,)), ((), ())),
            preferred_element_type=jnp.float32)
        local_min = jnp.min(scores, axis=1, keepdims=True)
        cand = jnp.where(scores == local_min, ids, jnp.int32(2**30))
        local_arg = jnp.min(cand, axis=1, keepdims=True) + j * _K_CHUNK
        upd = local_min < best_val
        best_val = jnp.where(upd, local_min, best_val)
        best_idx = jnp.where(upd, local_arg, best_idx)

    idx_ref[0, 0, :] = best_idx[:, 0]
    total = jnp.sum(best_val + zs).reshape(1, 1)
    prev = jnp.where(i == 0, jnp.zeros((1, 1), jnp.float32), loss_ref[...])
    loss_ref[...] = prev + total


def _vq_argmin(z_flat, codebook):
    m = z_flat.shape[0]
    n_blocks = 8
    bm = m // n_blocks
    idx3, loss = pl.pallas_call(
        _vq_body,
        grid=(n_blocks,),
        in_specs=[
            pl.BlockSpec((bm, _EMB_DIM), lambda i: (i, 0)),
            pl.BlockSpec((_NUM_EMB, _EMB_DIM), lambda i: (0, 0)),
        ],
        out_specs=[
            pl.BlockSpec((1, 1, bm), lambda i: (i, 0, 0)),
            pl.BlockSpec((1, 1), lambda i: (0, 0)),
        ],
        out_shape=[
            jax.ShapeDtypeStruct((n_blocks, 1, bm), jnp.int32),
            jax.ShapeDtypeStruct((1, 1), jnp.float32),
        ],
    )(z_flat, codebook)
    return idx3.reshape(m), loss[0, 0]


def _sc_gather(codebook, idx_padded, n_padded):
    n_workers = 32
    rows_per_w = n_padded // n_workers
    mesh = plsc.VectorSubcoreMesh(core_axis_name="c", subcore_axis_name="s")

    @functools.partial(
        pl.kernel,
        out_type=jax.ShapeDtypeStruct((n_padded, _EMB_DIM), jnp.float32),
        mesh=mesh,
        scratch_types=[
            pltpu.VMEM((rows_per_w,), jnp.int32),
            pltpu.VMEM((rows_per_w, _EMB_DIM), jnp.float32),
            pltpu.SemaphoreType.DMA,
        ],
        compiler_params=pltpu.CompilerParams(use_tc_tiling_on_sc=False),
    )
    def gather_kernel(table_hbm, idx_hbm, out_hbm, idx_v, rows_v, sem):
        wid = lax.axis_index("s") * 2 + lax.axis_index("c")
        base = wid * rows_per_w
        pltpu.sync_copy(idx_hbm.at[pl.ds(base, rows_per_w)], idx_v)
        pltpu.async_copy(table_hbm.at[idx_v], rows_v, sem).wait()
        pltpu.sync_copy(rows_v, out_hbm.at[pl.ds(base, rows_per_w)])

    return gather_kernel(codebook, idx_padded)


def kernel(x, enc_w1, enc_b1, enc_w2, enc_b2, enc_w3, enc_b3, codebook,
           dec_w1, dec_b1, dec_w2, dec_b2, dec_w3, dec_b3):
    z = jax.nn.relu(_conv2d(x.transpose(0, 2, 3, 1), enc_w1, enc_b1, 2))
    z = jax.nn.relu(_conv2d(z, enc_w2, enc_b2, 2))
    z_e = _conv2d(z, enc_w3, enc_b3, 2)
    B, H, W, C = z_e.shape
    m = B * H * W
    z_flat = z_e.reshape(m, C)

    indices, loss_sum = _vq_argmin(z_flat, codebook)

    n_padded = ((m + 255) // 256) * 256
    idx_padded = jnp.concatenate(
        [indices, jnp.zeros((n_padded - m,), jnp.int32)])
    z_q_flat = _sc_gather(codebook, idx_padded, n_padded)[:m]

    z_q = z_q_flat.reshape(B, H, W, C)

    r = jax.nn.relu(_conv2d_t(z_q, dec_w1, dec_b1))
    r = jax.nn.relu(_conv2d_t(r, dec_w2, dec_b2))
    x_recon = jax.nn.sigmoid(_conv2d_t(r, dec_w3, dec_b3))
    x_recon = x_recon.transpose(0, 3, 1, 2)

    loss = 1.25 * loss_sum / jnp.float32(m * C)
    return (x_recon, loss)


# speedup vs baseline: 1.0305x; 1.0221x over previous
import functools

import jax
import jax.numpy as jnp
from jax import lax
from jax.experimental import pallas as pl
from jax.experimental.pallas import tpu as pltpu
from jax.experimental.pallas import tpu_sc as plsc

_NUM_EMB = 8192
_EMB_DIM = 64
_K_CHUNK = 1024


def _conv2d(x, w, b, stride):
    y = lax.conv_general_dilated(
        x.astype(jnp.bfloat16),
        w.transpose(2, 3, 1, 0).astype(jnp.bfloat16), (stride, stride),
        ((1, 1), (1, 1)), dimension_numbers=('NHWC', 'HWIO', 'NHWC'),
        preferred_element_type=jnp.float32)
    return y + b[None, None, None, :]


def _conv2d_t(x, w, b):
    wt = jnp.flip(w, axis=(2, 3)).transpose(2, 3, 0, 1)
    y = lax.conv_general_dilated(
        x.astype(jnp.bfloat16), wt.astype(jnp.bfloat16), (1, 1),
        ((2, 2), (2, 2)), lhs_dilation=(2, 2),
        dimension_numbers=('NHWC', 'HWIO', 'NHWC'),
        preferred_element_type=jnp.float32)
    return y + b[None, None, None, :]


def _vq_body(z_ref, cb_ref, idx_ref, loss_ref):
    i = pl.program_id(0)
    bm = z_ref.shape[0]
    z = z_ref[...]
    zs = jnp.sum(z * z, axis=1, keepdims=True)
    z_bf = z.astype(jnp.bfloat16)
    ids = lax.broadcasted_iota(jnp.int32, (bm, _K_CHUNK), 1)

    best_val = jnp.full((bm, 1), jnp.inf, jnp.float32)
    best_idx = jnp.zeros((bm, 1), jnp.int32)
    for j in range(_NUM_EMB // _K_CHUNK):
        c = cb_ref[pl.ds(j * _K_CHUNK, _K_CHUNK), :]
        cs = jnp.sum(c * c, axis=1, keepdims=True)
        scores = cs.T - 2.0 * lax.dot_general(
            z_bf, c.astype(jnp.bfloat16), (((1,), (1,)), ((), ())),
            preferred_element_type=jnp.float32)
        local_min = jnp.min(scores, axis=1, keepdims=True)
        cand = jnp.where(scores == local_min, ids, jnp.int32(2**30))
        local_arg = jnp.min(cand, axis=1, keepdims=True) + j * _K_CHUNK
        upd = local_min < best_val
        best_val = jnp.where(upd, local_min, best_val)
        best_idx = jnp.where(upd, local_arg, best_idx)

    idx_ref[0, 0, :] = best_idx[:, 0]
    total = jnp.sum(best_val + zs).reshape(1, 1)
    prev = jnp.where(i == 0, jnp.zeros((1, 1), jnp.float32), loss_ref[...])
    loss_ref[...] = prev + total


def _vq_argmin(z_flat, codebook):
    m = z_flat.shape[0]
    n_blocks = 8
    bm = m // n_blocks
    idx3, loss = pl.pallas_call(
        _vq_body,
        grid=(n_blocks,),
        in_specs=[
            pl.BlockSpec((bm, _EMB_DIM), lambda i: (i, 0)),
            pl.BlockSpec((_NUM_EMB, _EMB_DIM), lambda i: (0, 0)),
        ],
        out_specs=[
            pl.BlockSpec((1, 1, bm), lambda i: (i, 0, 0)),
            pl.BlockSpec((1, 1), lambda i: (0, 0)),
        ],
        out_shape=[
            jax.ShapeDtypeStruct((n_blocks, 1, bm), jnp.int32),
            jax.ShapeDtypeStruct((1, 1), jnp.float32),
        ],
    )(z_flat, codebook)
    return idx3.reshape(m), loss[0, 0]


def _sc_gather(codebook, idx_padded, n_padded):
    n_workers = 32
    rows_per_w = n_padded // n_workers
    mesh = plsc.VectorSubcoreMesh(core_axis_name="c", subcore_axis_name="s")

    @functools.partial(
        pl.kernel,
        out_type=jax.ShapeDtypeStruct((n_padded, _EMB_DIM), jnp.float32),
        mesh=mesh,
        scratch_types=[
            pltpu.VMEM((rows_per_w,), jnp.int32),
            pltpu.VMEM((rows_per_w, _EMB_DIM), jnp.float32),
            pltpu.SemaphoreType.DMA,
        ],
        compiler_params=pltpu.CompilerParams(use_tc_tiling_on_sc=False),
    )
    def gather_kernel(table_hbm, idx_hbm, out_hbm, idx_v, rows_v, sem):
        wid = lax.axis_index("s") * 2 + lax.axis_index("c")
        base = wid * rows_per_w
        pltpu.sync_copy(idx_hbm.at[pl.ds(base, rows_per_w)], idx_v)
        pltpu.async_copy(table_hbm.at[idx_v], rows_v, sem).wait()
        pltpu.sync_copy(rows_v, out_hbm.at[pl.ds(base, rows_per_w)])

    return gather_kernel(codebook, idx_padded)


def kernel(x, enc_w1, enc_b1, enc_w2, enc_b2, enc_w3, enc_b3, codebook,
           dec_w1, dec_b1, dec_w2, dec_b2, dec_w3, dec_b3):
    z = jax.nn.relu(_conv2d(x.transpose(0, 2, 3, 1), enc_w1, enc_b1, 2))
    z = jax.nn.relu(_conv2d(z, enc_w2, enc_b2, 2))
    z_e = _conv2d(z, enc_w3, enc_b3, 2)
    B, H, W, C = z_e.shape
    m = B * H * W
    z_flat = z_e.reshape(m, C)

    indices, loss_sum = _vq_argmin(z_flat, codebook)

    n_padded = ((m + 255) // 256) * 256
    idx_padded = jnp.concatenate(
        [indices, jnp.zeros((n_padded - m,), jnp.int32)])
    z_q_flat = _sc_gather(codebook, idx_padded, n_padded)[:m]

    z_q = z_q_flat.reshape(B, H, W, C)

    r = jax.nn.relu(_conv2d_t(z_q, dec_w1, dec_b1))
    r = jax.nn.relu(_conv2d_t(r, dec_w2, dec_b2))
    x_recon = jax.nn.sigmoid(_conv2d_t(r, dec_w3, dec_b3))
    x_recon = x_recon.transpose(0, 3, 1, 2)

    loss = 1.25 * loss_sum / jnp.float32(m * C)
    return (x_recon, loss)
